# baseline (speedup 1.0000x reference)
import jax
import jax.numpy as jnp
from jax.experimental import pallas as pl
from jax.experimental.pallas import tpu as pltpu

_MARGIN = 0.5
_MARGIN_POS = 0.6
_NEG_TO_POS_RATIO = 3
_LOC_WEIGHT = 0.2

_B = 8
_L = 64
_A = 4096


def _neg_loss(cp_ref, ct_ref):
    ct = ct_ref[0]
    cp = cp_ref[0]
    pos = ct > 0
    neg = jnp.logical_not(jnp.logical_or(pos, ct == -1))
    vneg = jnp.where(neg, jnp.maximum(cp - _MARGIN, 0.0), 0.0)
    return pos, vneg * vneg


def _body(lp_ref, lt_ref, cp_ref, ct_ref, out_ref,
          locs_r, clsp_r, nposg_r, clsn_r):
    r = pl.program_id(0)

    @pl.when(r == 0)
    def _init():
        locs_r[0] = 0.0
        clsp_r[0] = 0.0
        nposg_r[0] = 0
        clsn_r[0] = 0.0

    pos, vneg = _neg_loss(cp_ref, ct_ref)
    cp = cp_ref[0]
    lpos = jnp.where(pos, jnp.maximum(_MARGIN_POS - cp, 0.0), 0.0)
    lpos = lpos * lpos

    npos = jnp.sum(pos.astype(jnp.float32))
    c = jnp.sum((vneg > 0.0).astype(jnp.float32))
    sumv = jnp.sum(vneg)
    clsp_r[0] += jnp.sum(lpos)
    nposg_r[0] += npos.astype(jnp.int32)

    d = lp_ref[0] - lt_ref[0]
    ad = jnp.abs(d)
    sl1 = jnp.where(ad < 1.0, 0.5 * d * d, ad - 0.5)
    row_i = jax.lax.broadcasted_iota(jnp.int32, (_L, 4 * _L), 0)
    col_j = jax.lax.broadcasted_iota(jnp.int32, (_L, 4 * _L), 1)
    s_mat = (jax.lax.shift_right_logical(col_j, 2) == row_i).astype(jnp.float32)
    sl1_sum = jax.lax.dot_general(
        s_mat, sl1, (((1,), (0,)), ((), ())),
        preferred_element_type=jnp.float32)
    locs_r[0] += jnp.sum(jnp.where(pos, sl1_sum, 0.0))

    k = _NEG_TO_POS_RATIO * npos.astype(jnp.int32)
    need_sel = jnp.logical_and(c.astype(jnp.int32) > k, k > 0)

    @pl.when(jnp.logical_not(need_sel))
    def _plain():
        clsn_r[0] += jnp.where(k == 0, 0.0, sumv)

    @pl.when(need_sel)
    def _select():
        def bit_step(i, prefix):
            cand = prefix | jax.lax.shift_left(jnp.int32(1), 30 - i)
            _, vv = _neg_loss(cp_ref, ct_ref)
            u = jax.lax.bitcast_convert_type(vv, jnp.int32)
            cnt = jnp.sum((u >= cand).astype(jnp.int32))
            return jnp.where(cnt >= k, cand, prefix)

        t = jax.lax.fori_loop(0, 31, bit_step, jnp.int32(0))
        _, v = _neg_loss(cp_ref, ct_ref)
        u = jax.lax.bitcast_convert_type(v, jnp.int32)
        gt = u > t
        ge = u >= t
        cnt_gt = jnp.sum(gt.astype(jnp.int32))
        cnt_ge = jnp.sum(ge.astype(jnp.int32))
        sum_gt = jnp.sum(jnp.where(gt, v, 0.0))
        sum_ge = jnp.sum(jnp.where(ge, v, 0.0))
        tf = (sum_ge - sum_gt) / (cnt_ge - cnt_gt).astype(jnp.float32)
        clsn_r[0] += sum_gt + (k - cnt_gt).astype(jnp.float32) * tf

    @pl.when(r == _B - 1)
    def _finish():
        denom = jnp.maximum(nposg_r[0].astype(jnp.float32), 1.0)
        cls_loss = (clsp_r[0] + clsn_r[0]) / denom
        loc_loss = locs_r[0] / denom
        out_ref[0] = cls_loss + _LOC_WEIGHT * loc_loss
        out_ref[1] = cls_loss
        out_ref[2] = loc_loss


def kernel(loc_preds, loc_targets, cls_preds, cls_targets):
    out = pl.pallas_call(
        _body,
        grid=(_B,),
        in_specs=[
            pl.BlockSpec((1, 4 * _L, _A), lambda r: (r, 0, 0)),
            pl.BlockSpec((1, 4 * _L, _A), lambda r: (r, 0, 0)),
            pl.BlockSpec((1, _L, _A), lambda r: (r, 0, 0)),
            pl.BlockSpec((1, _L, _A), lambda r: (r, 0, 0)),
        ],
        out_specs=pl.BlockSpec(memory_space=pltpu.SMEM),
        out_shape=jax.ShapeDtypeStruct((3,), jnp.float32),
        scratch_shapes=[
            pltpu.SMEM((1,), jnp.float32),
            pltpu.SMEM((1,), jnp.float32),
            pltpu.SMEM((1,), jnp.int32),
            pltpu.SMEM((1,), jnp.float32),
        ],
    )(loc_preds.reshape(_B, 4 * _L, _A), loc_targets.reshape(_B, 4 * _L, _A),
      cls_preds, cls_targets.astype(jnp.int32))
    return out[0], out[1], out[2]


# speedup vs baseline: 1.9490x; 1.9490x over previous
import jax
import jax.numpy as jnp
from jax.experimental import pallas as pl
from jax.experimental.pallas import tpu as pltpu

_MARGIN = 0.5
_MARGIN_POS = 0.6
_NEG_TO_POS_RATIO = 3
_LOC_WEIGHT = 0.2

_B = 8
_L = 64
_A = 4096


def _neg_loss(cp_ref, ct_ref):
    ct = ct_ref[0]
    cp = cp_ref[0]
    pos = ct > 0
    neg = jnp.logical_not(jnp.logical_or(pos, ct == -1))
    vneg = jnp.where(neg, jnp.maximum(cp - _MARGIN, 0.0), 0.0)
    return pos, vneg * vneg


def _body(lp_ref, lt_ref, cp_ref, ct_ref, out_ref,
          locs_r, clsp_r, nposg_r, clsn_r):
    r = pl.program_id(0)

    @pl.when(r == 0)
    def _init():
        locs_r[0] = 0.0
        clsp_r[0] = 0.0
        nposg_r[0] = 0
        clsn_r[0] = 0.0

    pos, vneg = _neg_loss(cp_ref, ct_ref)
    cp = cp_ref[0]
    lpos = jnp.where(pos, jnp.maximum(_MARGIN_POS - cp, 0.0), 0.0)
    lpos = lpos * lpos

    npos = jnp.sum(pos.astype(jnp.float32))
    c = jnp.sum((vneg > 0.0).astype(jnp.float32))
    sumv = jnp.sum(vneg)
    clsp_r[0] += jnp.sum(lpos)
    nposg_r[0] += npos.astype(jnp.int32)

    d = lp_ref[0] - lt_ref[0]
    ad = jnp.abs(d)
    sl1 = jnp.where(ad < 1.0, 0.5 * d * d, ad - 0.5).sum(axis=1)
    locs_r[0] += jnp.sum(jnp.where(pos, sl1, 0.0))

    k = _NEG_TO_POS_RATIO * npos.astype(jnp.int32)
    need_sel = jnp.logical_and(c.astype(jnp.int32) > k, k > 0)

    @pl.when(jnp.logical_not(need_sel))
    def _plain():
        clsn_r[0] += jnp.where(k == 0, 0.0, sumv)

    @pl.when(need_sel)
    def _select():
        def bit_step(i, prefix):
            cand = prefix | jax.lax.shift_left(jnp.int32(1), 30 - i)
            _, vv = _neg_loss(cp_ref, ct_ref)
            u = jax.lax.bitcast_convert_type(vv, jnp.int32)
            cnt = jnp.sum((u >= cand).astype(jnp.int32))
            return jnp.where(cnt >= k, cand, prefix)

        t = jax.lax.fori_loop(0, 31, bit_step, jnp.int32(0))
        _, v = _neg_loss(cp_ref, ct_ref)
        u = jax.lax.bitcast_convert_type(v, jnp.int32)
        gt = u > t
        ge = u >= t
        cnt_gt = jnp.sum(gt.astype(jnp.int32))
        cnt_ge = jnp.sum(ge.astype(jnp.int32))
        sum_gt = jnp.sum(jnp.where(gt, v, 0.0))
        sum_ge = jnp.sum(jnp.where(ge, v, 0.0))
        tf = (sum_ge - sum_gt) / (cnt_ge - cnt_gt).astype(jnp.float32)
        clsn_r[0] += sum_gt + (k - cnt_gt).astype(jnp.float32) * tf

    @pl.when(r == _B - 1)
    def _finish():
        denom = jnp.maximum(nposg_r[0].astype(jnp.float32), 1.0)
        cls_loss = (clsp_r[0] + clsn_r[0]) / denom
        loc_loss = locs_r[0] / denom
        out_ref[0] = cls_loss + _LOC_WEIGHT * loc_loss
        out_ref[1] = cls_loss
        out_ref[2] = loc_loss


def kernel(loc_preds, loc_targets, cls_preds, cls_targets):
    out = pl.pallas_call(
        _body,
        grid=(_B,),
        in_specs=[
            pl.BlockSpec((1, _L, 4, _A), lambda r: (r, 0, 0, 0)),
            pl.BlockSpec((1, _L, 4, _A), lambda r: (r, 0, 0, 0)),
            pl.BlockSpec((1, _L, _A), lambda r: (r, 0, 0)),
            pl.BlockSpec((1, _L, _A), lambda r: (r, 0, 0)),
        ],
        out_specs=pl.BlockSpec(memory_space=pltpu.SMEM),
        out_shape=jax.ShapeDtypeStruct((3,), jnp.float32),
        scratch_shapes=[
            pltpu.SMEM((1,), jnp.float32),
            pltpu.SMEM((1,), jnp.float32),
            pltpu.SMEM((1,), jnp.int32),
            pltpu.SMEM((1,), jnp.float32),
        ],
    )(loc_preds, loc_targets, cls_preds, cls_targets.astype(jnp.int32))
    return out[0], out[1], out[2]


# speedup vs baseline: 2.5679x; 1.3175x over previous
import jax
import jax.numpy as jnp
from jax.experimental import pallas as pl
from jax.experimental.pallas import tpu as pltpu

_MARGIN = 0.5
_MARGIN_POS = 0.6
_NEG_TO_POS_RATIO = 3
_LOC_WEIGHT = 0.2

_B = 8
_L = 64
_A = 4096


def _neg_loss(cp_ref, ct_ref):
    ct = ct_ref[0]
    cp = cp_ref[0]
    pos = ct > 0
    neg = jnp.logical_not(jnp.logical_or(pos, ct == -1))
    vneg = jnp.where(neg, jnp.maximum(cp - _MARGIN, 0.0), 0.0)
    return pos, vneg * vneg


def _body(lp_ref, lt_ref, cp_ref, ct_ref, out_ref,
          locs_r, clsp_r, nposg_r, clsn_r):
    r = pl.program_id(0)

    @pl.when(r == 0)
    def _init():
        locs_r[0] = 0.0
        clsp_r[0] = 0.0
        nposg_r[0] = 0
        clsn_r[0] = 0.0

    pos = ct_ref[0] > 0

    for j in range(4):
        d = lp_ref[0, :, j, :] - lt_ref[0, :, j, :]
        ad = jnp.abs(d)
        m = jnp.minimum(ad, 1.0)
        sl1 = m * (ad - 0.5 * m)
        locs_r[0] += jnp.sum(jnp.where(pos, sl1, 0.0))

    if True:
        _, vneg = _neg_loss(cp_ref, ct_ref)
        cp = cp_ref[0]
        lpos = jnp.where(pos, jnp.maximum(_MARGIN_POS - cp, 0.0), 0.0)
        lpos = lpos * lpos

        npos = jnp.sum(pos.astype(jnp.float32))
        c = jnp.sum((vneg > 0.0).astype(jnp.float32))
        sumv = jnp.sum(vneg)
        clsp_r[0] += jnp.sum(lpos)
        nposg_r[0] += npos.astype(jnp.int32)

        k = _NEG_TO_POS_RATIO * npos.astype(jnp.int32)
        need_sel = jnp.logical_and(c.astype(jnp.int32) > k, k > 0)

        @pl.when(jnp.logical_not(need_sel))
        def _plain():
            clsn_r[0] += jnp.where(k == 0, 0.0, sumv)

        @pl.when(need_sel)
        def _select():
            def bit_step(i, prefix):
                cand = prefix | jax.lax.shift_left(jnp.int32(1), 30 - i)
                _, vv = _neg_loss(cp_ref, ct_ref)
                u = jax.lax.bitcast_convert_type(vv, jnp.int32)
                cnt = jnp.sum((u >= cand).astype(jnp.int32))
                return jnp.where(cnt >= k, cand, prefix)

            t = jax.lax.fori_loop(0, 31, bit_step, jnp.int32(0))
            _, v = _neg_loss(cp_ref, ct_ref)
            u = jax.lax.bitcast_convert_type(v, jnp.int32)
            gt = u > t
            ge = u >= t
            cnt_gt = jnp.sum(gt.astype(jnp.int32))
            cnt_ge = jnp.sum(ge.astype(jnp.int32))
            sum_gt = jnp.sum(jnp.where(gt, v, 0.0))
            sum_ge = jnp.sum(jnp.where(ge, v, 0.0))
            tf = (sum_ge - sum_gt) / (cnt_ge - cnt_gt).astype(jnp.float32)
            clsn_r[0] += sum_gt + (k - cnt_gt).astype(jnp.float32) * tf

    @pl.when(r == _B - 1)
    def _finish():
        denom = jnp.maximum(nposg_r[0].astype(jnp.float32), 1.0)
        cls_loss = (clsp_r[0] + clsn_r[0]) / denom
        loc_loss = locs_r[0] / denom
        out_ref[0] = cls_loss + _LOC_WEIGHT * loc_loss
        out_ref[1] = cls_loss
        out_ref[2] = loc_loss


def kernel(loc_preds, loc_targets, cls_preds, cls_targets):
    out = pl.pallas_call(
        _body,
        grid=(_B,),
        in_specs=[
            pl.BlockSpec((1, _L, 4, _A), lambda r: (r, 0, 0, 0)),
            pl.BlockSpec((1, _L, 4, _A), lambda r: (r, 0, 0, 0)),
            pl.BlockSpec((1, _L, _A), lambda r: (r, 0, 0)),
            pl.BlockSpec((1, _L, _A), lambda r: (r, 0, 0)),
        ],
        out_specs=pl.BlockSpec(memory_space=pltpu.SMEM),
        out_shape=jax.ShapeDtypeStruct((3,), jnp.float32),
        scratch_shapes=[
            pltpu.SMEM((1,), jnp.float32),
            pltpu.SMEM((1,), jnp.float32),
            pltpu.SMEM((1,), jnp.int32),
            pltpu.SMEM((1,), jnp.float32),
        ],
    )(loc_preds, loc_targets, cls_preds, cls_targets.astype(jnp.int32))
    return out[0], out[1], out[2]
